# Initial kernel scaffold; baseline (speedup 1.0000x reference)
import jax
import jax.numpy as jnp
from jax.experimental import pallas as pl


def kernel(x, indices):
    raise NotImplementedError("write your pallas kernel here")


# speedup vs baseline: 1.9153x; 1.9153x over previous
import jax
import jax.numpy as jnp
from jax import lax
from jax.experimental import pallas as pl
from jax.experimental.pallas import tpu as pltpu
from jax.experimental.pallas import tpu_sc as plsc

_B, _L, _D = 16, 4096, 256
_LANES = 16
_NW = 32
_DBLKS = _D // _LANES
_TASKS = _B * _DBLKS
_TPW = _TASKS // _NW
_CHUNK = 1024
_NCHUNK = _L // _CHUNK


def _body(x_hbm, idx_hbm, out_hbm, x_v, idx_v, out_v):
    cid = lax.axis_index("c")
    sid = lax.axis_index("s")
    wid = sid * 2 + cid
    lanes = lax.iota(jnp.int32, _LANES)

    def task(i, carry):
        t = wid + i * _NW
        b = t // _DBLKS
        d0 = (t % _DBLKS) * _LANES
        pltpu.sync_copy(x_hbm.at[b, :, pl.ds(d0, _LANES)], x_v)

        def chunk(ci, carry2):
            l0 = ci * _CHUNK
            pltpu.sync_copy(idx_hbm.at[b, pl.ds(l0, _CHUNK), pl.ds(d0, _LANES)],
                            idx_v)

            def row(l, carry3):
                iv = idx_v[l, :]
                out_v[l, :] = plsc.load_gather(x_v, [iv, lanes])
                return carry3

            lax.fori_loop(0, _CHUNK, row, 0)
            pltpu.sync_copy(out_v,
                            out_hbm.at[b, pl.ds(l0, _CHUNK), pl.ds(d0, _LANES)])
            return carry2

        lax.fori_loop(0, _NCHUNK, chunk, 0)
        return carry

    lax.fori_loop(0, _TPW, task, 0)


def kernel(x, indices):
    run = pl.kernel(
        _body,
        out_type=jax.ShapeDtypeStruct((_B, _L, _D), jnp.float32),
        mesh=plsc.VectorSubcoreMesh(core_axis_name="c", subcore_axis_name="s"),
        compiler_params=pltpu.CompilerParams(
            use_tc_tiling_on_sc=False, needs_layout_passes=False),
        scratch_types=[
            pltpu.VMEM((_L, _LANES), jnp.float32),
            pltpu.VMEM((_CHUNK, _LANES), jnp.int32),
            pltpu.VMEM((_CHUNK, _LANES), jnp.float32),
        ],
    )
    return run(x, indices)


# speedup vs baseline: 2.3977x; 1.2519x over previous
import jax
import jax.numpy as jnp
from jax import lax
from jax.experimental import pallas as pl
from jax.experimental.pallas import tpu as pltpu
from jax.experimental.pallas import tpu_sc as plsc

_B, _L, _D = 16, 4096, 256
_LANES = 16
_NW = 32
_DBLKS = _D // _LANES
_TASKS = _B * _DBLKS
_TPW = _TASKS // _NW
_CHUNK = 1024
_NCHUNK = _L // _CHUNK


def _body(x_hbm, idx_hbm, out_hbm, x_v, idx_v, out_v):
    cid = lax.axis_index("c")
    sid = lax.axis_index("s")
    wid = sid * 2 + cid
    lanes = lax.iota(jnp.int32, _LANES)

    def task(i, carry):
        t = wid + i * _NW
        b = t // _DBLKS
        d0 = (t % _DBLKS) * _LANES
        pltpu.sync_copy(x_hbm.at[b, :, pl.ds(d0, _LANES)], x_v)

        def chunk(ci, carry2):
            l0 = ci * _CHUNK
            pltpu.sync_copy(idx_hbm.at[b, pl.ds(l0, _CHUNK), pl.ds(d0, _LANES)],
                            idx_v)

            @plsc.parallel_loop(0, _CHUNK, unroll=8)
            def _row(l):
                iv = idx_v[l, :]
                out_v[l, :] = plsc.load_gather(x_v, [iv, lanes])
            pltpu.sync_copy(out_v,
                            out_hbm.at[b, pl.ds(l0, _CHUNK), pl.ds(d0, _LANES)])
            return carry2

        lax.fori_loop(0, _NCHUNK, chunk, 0)
        return carry

    lax.fori_loop(0, _TPW, task, 0)


def kernel(x, indices):
    run = pl.kernel(
        _body,
        out_type=jax.ShapeDtypeStruct((_B, _L, _D), jnp.float32),
        mesh=plsc.VectorSubcoreMesh(core_axis_name="c", subcore_axis_name="s"),
        compiler_params=pltpu.CompilerParams(
            use_tc_tiling_on_sc=False, needs_layout_passes=False),
        scratch_types=[
            pltpu.VMEM((_L, _LANES), jnp.float32),
            pltpu.VMEM((_CHUNK, _LANES), jnp.int32),
            pltpu.VMEM((_CHUNK, _LANES), jnp.float32),
        ],
    )
    return run(x, indices)
